# baseline (speedup 1.0000x reference)
import functools

import jax
import jax.numpy as jnp
from jax import lax
from jax.experimental import pallas as pl
from jax.experimental.pallas import tpu as pltpu
from jax.experimental.pallas import tpu_sc as plsc

N = 100000
DEG = 32
EMB = 128
SFW = 256
NSTEP = 5
WIN = 128
TAIL = N - WIN


def _win_base(root):
    return pl.multiple_of(
        jnp.minimum((root // WIN) * WIN, ((N - WIN) // WIN) * WIN), WIN)


def _ids_body(path_smem, ntT_hbm, tail_ref, ids_ref, win_v, sem):
    for s in range(NSTEP):
        root = path_smem[2 * s]
        pltpu.make_async_copy(
            ntT_hbm.at[:, pl.ds(_win_base(root), WIN)], win_v.at[s],
            sem).start()
    tail = tail_ref[...]
    for s in range(NSTEP):
        root = path_smem[2 * s]
        pltpu.make_async_copy(
            ntT_hbm.at[:, pl.ds(_win_base(root), WIN)], win_v.at[s],
            sem).wait()
        in_tail = root >= TAIL
        lane = root - jnp.where(in_tail, TAIL, _win_base(root))
        mask = lax.broadcasted_iota(jnp.int32, (DEG, WIN), 1) == lane
        block = jnp.where(in_tail, tail, win_v[s])
        ids_i = jnp.sum(jnp.where(mask, block, 0), axis=1).reshape(1, DEG)
        row = jnp.concatenate(
            [jnp.full((1, 1), root, jnp.int32), ids_i,
             jnp.zeros((1, 128 - 1 - DEG), jnp.int32)], axis=1)
        ids_ref[pl.ds(s, 1), :] = row


_ids_lookup = pl.pallas_call(
    _ids_body,
    in_specs=[pl.BlockSpec(memory_space=pltpu.SMEM),
              pl.BlockSpec(memory_space=pl.ANY),
              pl.BlockSpec(memory_space=pltpu.VMEM)],
    out_shape=jax.ShapeDtypeStruct((8, 128), jnp.int32),
    scratch_shapes=[pltpu.VMEM((NSTEP, DEG, WIN), jnp.int32),
                    pltpu.SemaphoreType.DMA],
    compiler_params=pltpu.CompilerParams(
        allow_input_fusion=[False, False, True]),
)


@functools.cache
def _make_sc_gather():
    mesh = plsc.VectorSubcoreMesh(core_axis_name="c", subcore_axis_name="s",
                                  num_cores=1)

    @functools.partial(
        pl.kernel,
        out_type=jax.ShapeDtypeStruct((8 * 2 * EMB,), jnp.float32),
        mesh=mesh,
        scratch_types=[
            pltpu.VMEM((128,), jnp.int32),
            pltpu.VMEM((1 + DEG, EMB), jnp.float32),
            pltpu.VMEM((2 * EMB,), jnp.float32),
            pltpu.SemaphoreType.DMA,
        ],
        compiler_params=pltpu.CompilerParams(
            use_tc_tiling_on_sc=False,
            disable_bounds_checks=True,
            disable_semaphore_checks=True),
    )
    def sc_gather(ids_hbm, emb_hbm, out_hbm, ids_v, rows_v, feat_v, sem):
        w = lax.axis_index("s")

        @pl.when(w < NSTEP)
        def _():
            pltpu.sync_copy(ids_hbm.at[w], ids_v)
            pltpu.async_copy(
                emb_hbm.at[ids_v.at[pl.ds(0, 1 + DEG)]], rows_v, sem).wait()
            for cch in range(EMB // 16):
                sl = pl.ds(16 * cch, 16)
                acc = rows_v[1, sl]
                for r in range(2, 1 + DEG):
                    acc = acc + rows_v[r, sl]
                feat_v[sl] = rows_v[0, sl]
                feat_v[pl.ds(EMB + 16 * cch, 16)] = acc * (1.0 / DEG)
            pltpu.sync_copy(feat_v, out_hbm.at[pl.ds(w * 2 * EMB, 2 * EMB)])

    return sc_gather


def _tc_body(feats_ref, wgc_ref, bgc_ref, wk_ref, wr_ref, bl_ref,
             w1t_ref, b1_ref, w2_ref, b2_ref, w3t_ref, b3_ref, out_ref):
    wgc = wgc_ref[...]
    bgc = bgc_ref[...]
    wk = wk_ref[...]
    wr = wr_ref[...]
    bl = bl_ref[...]
    h = jnp.zeros((1, SFW), jnp.float32)
    c = jnp.zeros((1, SFW), jnp.float32)
    for s in range(NSTEP):
        feat = feats_ref[pl.ds(s * 2 * EMB, 2 * EMB)].reshape(1, 2 * EMB)
        ent = jnp.maximum(
            jnp.dot(feat, wgc, preferred_element_type=jnp.float32) + bgc, 0.0)
        z = (jnp.dot(ent, wk, preferred_element_type=jnp.float32)
             + jnp.dot(h, wr, preferred_element_type=jnp.float32) + bl)
        i_g = jax.nn.sigmoid(z[:, 0:SFW])
        f_g = jax.nn.sigmoid(z[:, SFW:2 * SFW])
        g_g = jnp.tanh(z[:, 2 * SFW:3 * SFW])
        o_g = jax.nn.sigmoid(z[:, 3 * SFW:4 * SFW])
        c = f_g * c + i_g * g_g
        h = o_g * jnp.tanh(c)
    dims = (((1,), (1,)), ((), ()))
    x = jnp.maximum(
        lax.dot_general(h, w1t_ref[...], dims,
                        preferred_element_type=jnp.float32) + b1_ref[...], 0.0)
    x = jnp.maximum(
        jnp.dot(x, w2_ref[...], preferred_element_type=jnp.float32)
        + b2_ref[...], 0.0)
    logits = (lax.dot_general(x, w3t_ref[...], dims,
                              preferred_element_type=jnp.float32)
              + b3_ref[...])
    m = jnp.max(logits)
    e = jnp.exp(logits - m)
    out_ref[...] = e / jnp.sum(e)


_tc_forward = pl.pallas_call(
    _tc_body,
    out_shape=jax.ShapeDtypeStruct((1, 2), jnp.float32),
)


def kernel(path, neighbor_table, emb, W_gc, b_gc, lstm_kernel, lstm_rec,
           lstm_bias, W1, b1, W2, b2, W3, b3):
    ntT = neighbor_table.T
    ids = _ids_lookup(path, ntT, lax.slice(ntT, (0, TAIL), (DEG, N)))
    feats = _make_sc_gather()(ids, emb)
    out = _tc_forward(feats, W_gc, b_gc[None, :], lstm_kernel, lstm_rec,
                      lstm_bias[None, :], W1.T, b1[None, :], W2,
                      b2[None, :], W3.T, b3[None, :])
    return out[0]


# speedup vs baseline: 1.0297x; 1.0297x over previous
import functools

import jax
import jax.numpy as jnp
from jax import lax
from jax.experimental import pallas as pl
from jax.experimental.pallas import tpu as pltpu
from jax.experimental.pallas import tpu_sc as plsc

N = 100000
DEG = 32
EMB = 128
SFW = 256
NSTEP = 5
WIN = 128
TAIL = N - WIN


def _win_base(root):
    return pl.multiple_of(
        jnp.minimum((root // WIN) * WIN, ((N - WIN) // WIN) * WIN), WIN)


def _ids_body(path_smem, ntT_hbm, tail_ref, ids_ref, win_v, sem):
    for s in range(NSTEP):
        root = path_smem[2 * s]
        pltpu.make_async_copy(
            ntT_hbm.at[:, pl.ds(_win_base(root), WIN)], win_v.at[s],
            sem).start()
    tail = tail_ref[...]
    for s in range(NSTEP):
        root = path_smem[2 * s]
        pltpu.make_async_copy(
            ntT_hbm.at[:, pl.ds(_win_base(root), WIN)], win_v.at[s],
            sem).wait()
        in_tail = root >= TAIL
        lane = root - jnp.where(in_tail, TAIL, _win_base(root))
        mask = lax.broadcasted_iota(jnp.int32, (DEG, WIN), 1) == lane
        block = jnp.where(in_tail, tail, win_v[s])
        ids_i = jnp.sum(jnp.where(mask, block, 0), axis=1).reshape(1, DEG)
        row = jnp.concatenate(
            [jnp.full((1, 1), root, jnp.int32), ids_i,
             jnp.zeros((1, 128 - 1 - DEG), jnp.int32)], axis=1)
        ids_ref[pl.ds(s, 1), :] = row


_ids_lookup = pl.pallas_call(
    _ids_body,
    in_specs=[pl.BlockSpec(memory_space=pltpu.SMEM),
              pl.BlockSpec(memory_space=pl.ANY),
              pl.BlockSpec(memory_space=pltpu.VMEM)],
    out_shape=jax.ShapeDtypeStruct((8, 128), jnp.int32),
    scratch_shapes=[pltpu.VMEM((NSTEP, DEG, WIN), jnp.int32),
                    pltpu.SemaphoreType.DMA],
    compiler_params=pltpu.CompilerParams(
        allow_input_fusion=[False, False, True]),
)


@functools.cache
def _make_sc_gather():
    mesh = plsc.VectorSubcoreMesh(core_axis_name="c", subcore_axis_name="s",
                                  num_cores=1)

    @functools.partial(
        pl.kernel,
        out_type=jax.ShapeDtypeStruct((8 * 2 * EMB,), jnp.float32),
        mesh=mesh,
        scratch_types=[
            pltpu.VMEM((128,), jnp.int32),
            pltpu.VMEM((1 + DEG, EMB), jnp.float32),
            pltpu.VMEM((2 * EMB,), jnp.float32),
            pltpu.SemaphoreType.DMA,
        ],
        compiler_params=pltpu.CompilerParams(use_tc_tiling_on_sc=False),
    )
    def sc_gather(ids_hbm, emb_hbm, out_hbm, ids_v, rows_v, feat_v, sem):
        w = lax.axis_index("s")

        @pl.when(w < NSTEP)
        def _():
            pltpu.sync_copy(ids_hbm.at[w], ids_v)
            pltpu.async_copy(
                emb_hbm.at[ids_v.at[pl.ds(0, 1 + DEG)]], rows_v, sem).wait()

            def _row(r, accs):
                return tuple(
                    acc + rows_v[r, pl.ds(16 * c, 16)]
                    for c, acc in enumerate(accs))

            accs = tuple(
                rows_v[1, pl.ds(16 * c, 16)] for c in range(EMB // 16))
            accs = lax.fori_loop(2, 1 + DEG, _row, accs)
            for c in range(EMB // 16):
                feat_v[pl.ds(16 * c, 16)] = rows_v[0, pl.ds(16 * c, 16)]
                feat_v[pl.ds(EMB + 16 * c, 16)] = accs[c] * (1.0 / DEG)
            pltpu.sync_copy(feat_v, out_hbm.at[pl.ds(w * 2 * EMB, 2 * EMB)])

    return sc_gather


def _tc_body(feats_ref, wgc_ref, bgc_ref, wk_ref, wr_ref, bl_ref,
             w1t_ref, b1_ref, w2_ref, b2_ref, w3t_ref, b3_ref, out_ref):
    wgc = wgc_ref[...]
    bgc = bgc_ref[...]
    wk = wk_ref[...]
    wr = wr_ref[...]
    bl = bl_ref[...]
    h = jnp.zeros((1, SFW), jnp.float32)
    c = jnp.zeros((1, SFW), jnp.float32)
    for s in range(NSTEP):
        feat = feats_ref[pl.ds(s * 2 * EMB, 2 * EMB)].reshape(1, 2 * EMB)
        ent = jnp.maximum(
            jnp.dot(feat, wgc, preferred_element_type=jnp.float32) + bgc, 0.0)
        z = (jnp.dot(ent, wk, preferred_element_type=jnp.float32)
             + jnp.dot(h, wr, preferred_element_type=jnp.float32) + bl)
        i_g = jax.nn.sigmoid(z[:, 0:SFW])
        f_g = jax.nn.sigmoid(z[:, SFW:2 * SFW])
        g_g = jnp.tanh(z[:, 2 * SFW:3 * SFW])
        o_g = jax.nn.sigmoid(z[:, 3 * SFW:4 * SFW])
        c = f_g * c + i_g * g_g
        h = o_g * jnp.tanh(c)
    dims = (((1,), (1,)), ((), ()))
    x = jnp.maximum(
        lax.dot_general(h, w1t_ref[...], dims,
                        preferred_element_type=jnp.float32) + b1_ref[...], 0.0)
    x = jnp.maximum(
        jnp.dot(x, w2_ref[...], preferred_element_type=jnp.float32)
        + b2_ref[...], 0.0)
    logits = (lax.dot_general(x, w3t_ref[...], dims,
                              preferred_element_type=jnp.float32)
              + b3_ref[...])
    m = jnp.max(logits)
    e = jnp.exp(logits - m)
    out_ref[...] = e / jnp.sum(e)


_tc_forward = pl.pallas_call(
    _tc_body,
    out_shape=jax.ShapeDtypeStruct((1, 2), jnp.float32),
)


def kernel(path, neighbor_table, emb, W_gc, b_gc, lstm_kernel, lstm_rec,
           lstm_bias, W1, b1, W2, b2, W3, b3):
    ntT = neighbor_table.T
    ids = _ids_lookup(path, ntT, lax.slice(ntT, (0, TAIL), (DEG, N)))
    feats = _make_sc_gather()(ids, emb)
    out = _tc_forward(feats, W_gc, b_gc[None, :], lstm_kernel, lstm_rec,
                      lstm_bias[None, :], W1.T, b1[None, :], W2,
                      b2[None, :], W3.T, b3[None, :])
    return out[0]
